# baseline (speedup 1.0000x reference)
import jax
import jax.numpy as jnp
from jax.experimental import pallas as pl
from jax.experimental.pallas import tpu as pltpu


def _gate_scale(H):
    lane = jax.lax.broadcasted_iota(jnp.int32, (1, 4 * H), 1)
    return jnp.where((lane < 2 * H) | (lane >= 3 * H), 0.5, 1.0)


def _lstm_steps(whb_sc, gx_sc, store, h, c):
    TC = gx_sc.shape[0]
    H = h.shape[1]
    for t in range(TC):
        g = jnp.dot(h, whb_sc[...],
                    preferred_element_type=jnp.float32) + gx_sc[t]
        t_if = jnp.tanh(g[:, :2 * H])
        t_g = jnp.tanh(g[:, 2 * H:3 * H])
        t_o = jnp.tanh(g[:, 3 * H:])
        c = 0.5 * (c + t_if[:, H:] * c + (1.0 + t_if[:, :H]) * t_g)
        h = (0.5 * (1.0 + t_o)) * jnp.tanh(c)
        store(t, h)
    return h, c


def _fused_kernel(idx_ref, emb_ref, w0x_ref, w0h_ref, b0_ref,
                  w1x_ref, w1h_ref, b1_ref, wd_ref, bd_ref, h0_ref, c0_ref,
                  logits_ref, hf0_ref, cf0_ref, hf1_ref, cf1_ref,
                  ew_sc, whb0_sc, wxb1_sc, whb1_sc, gx0_sc, gx1_sc,
                  h0buf_sc, hd_sc, h0_sc, c0_sc, h1_sc, c1_sc):
    TC, B, H = hd_sc.shape
    V = emb_ref.shape[0]
    G = 4 * H
    j = pl.program_id(0)
    NT = pl.num_programs(0) - 1
    scale = _gate_scale(H)

    @pl.when(j == 0)
    def _():
        ew_sc[...] = jnp.dot(emb_ref[...], w0x_ref[0],
                             preferred_element_type=jnp.float32) * scale
        whb0_sc[...] = (w0h_ref[0] * scale).astype(jnp.bfloat16)
        wxb1_sc[...] = (w1x_ref[0] * scale).astype(jnp.bfloat16)
        whb1_sc[...] = (w1h_ref[0] * scale).astype(jnp.bfloat16)
        h0_sc[...] = h0_ref[0, 0]
        c0_sc[...] = c0_ref[0, 0]
        h1_sc[...] = h0_ref[0, 1]
        c1_sc[...] = c0_ref[0, 1]

    cur = jax.lax.rem(j, 2)
    prv = jax.lax.rem(j + 1, 2)

    idx = idx_ref[0]
    iota_v = jax.lax.broadcasted_iota(jnp.int32, (V, TC * B), 0)
    oh_t = (iota_v == idx).astype(jnp.float32)
    gx0_sc[...] = (jax.lax.dot_general(
        oh_t, ew_sc[...],
        dimension_numbers=(((0,), (0,)), ((), ())),
        preferred_element_type=jnp.float32)
        + b0_ref[0] * scale).reshape(TC, B, G)

    def store0(t, h):
        h0buf_sc[cur, t] = h
    h0v, c0v = _lstm_steps(whb0_sc, gx0_sc, store0, h0_sc[...], c0_sc[...])

    x1 = h0buf_sc[prv].reshape(TC * B, H)
    gx1_sc[...] = (jnp.dot(x1, wxb1_sc[...],
                           preferred_element_type=jnp.float32)
                   + b1_ref[0] * scale).reshape(TC, B, G)

    def store1(t, h):
        hd_sc[t] = h
    h1v, c1v = _lstm_steps(whb1_sc, gx1_sc, store1, h1_sc[...], c1_sc[...])

    logits_ref[...] = jnp.dot(hd_sc[...].reshape(TC * B, H), wd_ref[0],
                              preferred_element_type=jnp.float32) + bd_ref[0]

    @pl.when(j < NT)
    def _():
        h0_sc[...] = h0v
        c0_sc[...] = c0v

    @pl.when(j > 0)
    def _():
        h1_sc[...] = h1v
        c1_sc[...] = c1v

    hf0_ref[...] = h0_sc[...]
    cf0_ref[...] = c0_sc[...]
    hf1_ref[...] = h1v
    cf1_ref[...] = c1v


def kernel(idx_seq, emb, w_all, b_all, h0, c0):
    T, B = idx_seq.shape
    V, H = emb.shape
    G = 4 * H
    O = 256
    TB = T * B
    TC = 16 if T % 16 == 0 else T
    NT = T // TC

    idx_r = idx_seq.astype(jnp.int32).reshape(NT, 1, TC * B)
    h0_r = h0.reshape(1, 2, B, H)
    c0_r = c0.reshape(1, 2, B, H)
    last = NT - 1

    logits, hf0, cf0, hf1, cf1 = pl.pallas_call(
        _fused_kernel,
        grid=(NT + 1,),
        in_specs=[
            pl.BlockSpec((1, 1, TC * B),
                         lambda j, n=last: (jnp.minimum(j, n), 0, 0)),
            pl.BlockSpec((V, H), lambda j: (0, 0)),
            pl.BlockSpec((1, H, G), lambda j: (0, 0, 0)),
            pl.BlockSpec((1, H, G), lambda j: (0, 1, 0)),
            pl.BlockSpec((1, 1, G), lambda j: (0, 0, 0)),
            pl.BlockSpec((1, H, G), lambda j: (1, 0, 0)),
            pl.BlockSpec((1, H, G), lambda j: (1, 1, 0)),
            pl.BlockSpec((1, 1, G), lambda j: (1, 0, 0)),
            pl.BlockSpec((1, H, O), lambda j: (2, 0, 0)),
            pl.BlockSpec((1, 1, O), lambda j: (2, 0, 0)),
            pl.BlockSpec((1, 2, B, H), lambda j: (0, 0, 0, 0)),
            pl.BlockSpec((1, 2, B, H), lambda j: (0, 0, 0, 0)),
        ],
        out_specs=[
            pl.BlockSpec((TC * B, O),
                         lambda j: (jnp.maximum(j - 1, 0), 0)),
            pl.BlockSpec((B, H), lambda j: (0, 0)),
            pl.BlockSpec((B, H), lambda j: (0, 0)),
            pl.BlockSpec((B, H), lambda j: (0, 0)),
            pl.BlockSpec((B, H), lambda j: (0, 0)),
        ],
        out_shape=[jax.ShapeDtypeStruct((TB, O), jnp.float32)]
                  + [jax.ShapeDtypeStruct((B, H), jnp.float32)] * 4,
        scratch_shapes=[
            pltpu.VMEM((V, G), jnp.float32),
            pltpu.VMEM((H, G), jnp.bfloat16),
            pltpu.VMEM((H, G), jnp.bfloat16),
            pltpu.VMEM((H, G), jnp.bfloat16),
            pltpu.VMEM((TC, B, G), jnp.float32),
            pltpu.VMEM((TC, B, G), jnp.float32),
            pltpu.VMEM((2, TC, B, H), jnp.float32),
            pltpu.VMEM((TC, B, H), jnp.float32),
            pltpu.VMEM((B, H), jnp.float32),
            pltpu.VMEM((B, H), jnp.float32),
            pltpu.VMEM((B, H), jnp.float32),
            pltpu.VMEM((B, H), jnp.float32),
        ],
        compiler_params=pltpu.CompilerParams(
            dimension_semantics=("arbitrary",)),
    )(idx_r, emb, w_all, w_all, b_all, w_all, w_all, b_all, w_all, b_all,
      h0_r, c0_r)

    h_n = jnp.stack([hf0, hf1])
    c_n = jnp.stack([cf0, cf1])
    return logits.reshape(T, B, O), (h_n, c_n)


# speedup vs baseline: 1.0963x; 1.0963x over previous
import jax
import jax.numpy as jnp
from jax.experimental import pallas as pl
from jax.experimental.pallas import tpu as pltpu


def _gate_scale(H):
    lane = jax.lax.broadcasted_iota(jnp.int32, (1, 4 * H), 1)
    return jnp.where((lane < 2 * H) | (lane >= 3 * H), 0.5, 1.0)


def _lstm_steps(wh_ref, gx_sc, hout_ref, h_sc, c_sc, TC):
    H = h_sc.shape[1]
    U = TC

    def group(gidx, carry):
        h, c = carry
        base = gidx * U
        for u in range(U):
            t = base + u
            g = jnp.dot(h, wh_ref[...],
                        preferred_element_type=jnp.float32) + gx_sc[t]
            t_if = jnp.tanh(g[:, :2 * H])
            t_g = jnp.tanh(g[:, 2 * H:3 * H])
            t_o = jnp.tanh(g[:, 3 * H:])
            c = 0.5 * (c + t_if[:, H:] * c + (1.0 + t_if[:, :H]) * t_g)
            h = (0.5 * (1.0 + t_o)) * jnp.tanh(c)
            hout_ref[t] = h
        return h, c

    h_f, c_f = jax.lax.fori_loop(0, TC // U, group, (h_sc[...], c_sc[...]))
    h_sc[...] = h_f
    c_sc[...] = c_f


def _l0_kernel(idx_ref, emb_ref, wx_ref, wh_ref, b_ref, h0_ref, c0_ref,
               hout_ref, cfin_ref, ew_sc, gx_sc, h_sc, c_sc, whb_sc):
    TC, Bf, H = hout_ref.shape
    V = emb_ref.shape[0]

    scale = _gate_scale(H)

    @pl.when(pl.program_id(0) == 0)
    def _():
        ew_sc[...] = jnp.dot(emb_ref[...], wx_ref[0],
                             preferred_element_type=jnp.float32) * scale
        whb_sc[...] = (wh_ref[0] * scale).astype(jnp.bfloat16)
        h_sc[...] = h0_ref[0]
        c_sc[...] = c0_ref[0]

    idx = idx_ref[0]
    iota_v = jax.lax.broadcasted_iota(jnp.int32, (V, TC * Bf), 0)
    oh_t = (iota_v == idx).astype(jnp.float32)
    gx = jax.lax.dot_general(
        oh_t, ew_sc[...],
        dimension_numbers=(((0,), (0,)), ((), ())),
        preferred_element_type=jnp.float32) + b_ref[0] * scale
    gx_sc[...] = gx.reshape(TC, Bf, 4 * H)

    _lstm_steps(whb_sc, gx_sc, hout_ref, h_sc, c_sc, TC)
    cfin_ref[...] = c_sc[...]


def _l1_kernel(hin_ref, wx_ref, wh_ref, b_ref, h0_ref, c0_ref,
               hout_ref, cfin_ref, gx_sc, h_sc, c_sc, whb_sc, wxb_sc):
    TC, Bf, H = hin_ref.shape
    scale = _gate_scale(H)

    @pl.when(pl.program_id(0) == 0)
    def _():
        whb_sc[...] = (wh_ref[0] * scale).astype(jnp.bfloat16)
        wxb_sc[...] = (wx_ref[0] * scale).astype(jnp.bfloat16)
        h_sc[...] = h0_ref[0]
        c_sc[...] = c0_ref[0]

    x = hin_ref[...].reshape(TC * Bf, H)
    gx = jnp.dot(x, wxb_sc[...],
                 preferred_element_type=jnp.float32) + b_ref[0] * scale
    gx_sc[...] = gx.reshape(TC, Bf, 4 * H)

    _lstm_steps(whb_sc, gx_sc, hout_ref, h_sc, c_sc, TC)
    cfin_ref[...] = c_sc[...]


def _dec_kernel(x_ref, w_ref, b_ref, o_ref):
    o_ref[...] = jnp.dot(x_ref[...], w_ref[0],
                         preferred_element_type=jnp.float32) + b_ref[0]


def kernel(idx_seq, emb, w_all, b_all, h0, c0):
    T, B = idx_seq.shape
    V, H = emb.shape
    G = 4 * H
    O = 256
    TB = T * B
    TC = 32 if T % 32 == 0 else T
    NT = T // TC

    idx_r = idx_seq.astype(jnp.int32).reshape(NT, 1, TC * B)

    def layer_specs(l):
        return [
            pl.BlockSpec((1, H, G), lambda j, l=l: (l, 0, 0)),
            pl.BlockSpec((1, H, G), lambda j, l=l: (l, 1, 0)),
            pl.BlockSpec((1, 1, G), lambda j, l=l: (l, 0, 0)),
            pl.BlockSpec((1, B, H), lambda j, l=l: (l, 0, 0)),
            pl.BlockSpec((1, B, H), lambda j, l=l: (l, 0, 0)),
        ]

    out_specs = [
        pl.BlockSpec((TC, B, H), lambda j: (j, 0, 0)),
        pl.BlockSpec((B, H), lambda j: (0, 0)),
    ]
    out_shape = [jax.ShapeDtypeStruct((T, B, H), jnp.float32),
                 jax.ShapeDtypeStruct((B, H), jnp.float32)]
    state_scratch = [pltpu.VMEM((TC, B, G), jnp.float32),
                     pltpu.VMEM((B, H), jnp.float32),
                     pltpu.VMEM((B, H), jnp.float32),
                     pltpu.VMEM((H, G), jnp.bfloat16)]
    seq_sem = pltpu.CompilerParams(dimension_semantics=("arbitrary",))

    h_all0, c_fin0 = pl.pallas_call(
        _l0_kernel,
        grid=(NT,),
        in_specs=[pl.BlockSpec((1, 1, TC * B), lambda j: (j, 0, 0)),
                  pl.BlockSpec((V, H), lambda j: (0, 0))] + layer_specs(0),
        out_specs=out_specs,
        out_shape=out_shape,
        scratch_shapes=[pltpu.VMEM((V, G), jnp.float32)] + state_scratch,
        compiler_params=seq_sem,
    )(idx_r, emb, w_all, w_all, b_all, h0, c0)

    h_all1, c_fin1 = pl.pallas_call(
        _l1_kernel,
        grid=(NT,),
        in_specs=[pl.BlockSpec((TC, B, H), lambda j: (j, 0, 0))]
                 + layer_specs(1),
        out_specs=out_specs,
        out_shape=out_shape,
        scratch_shapes=state_scratch + [pltpu.VMEM((H, G), jnp.bfloat16)],
        compiler_params=seq_sem,
    )(h_all0, w_all, w_all, b_all, h0, c0)

    MBd = TB // 4
    logits = pl.pallas_call(
        _dec_kernel,
        grid=(2, 2),
        in_specs=[
            pl.BlockSpec((MBd, H), lambda bi, j: (bi * 2 + j, 0)),
            pl.BlockSpec((1, H, O), lambda bi, j: (2, 0, 0)),
            pl.BlockSpec((1, 1, O), lambda bi, j: (2, 0, 0)),
        ],
        out_specs=pl.BlockSpec((MBd, O), lambda bi, j: (bi * 2 + j, 0)),
        out_shape=jax.ShapeDtypeStruct((TB, O), jnp.float32),
        compiler_params=pltpu.CompilerParams(
            dimension_semantics=("parallel", "arbitrary")),
    )(h_all1.reshape(TB, H), w_all, b_all)

    h_n = jnp.stack([h_all0[T - 1], h_all1[T - 1]])
    c_n = jnp.stack([c_fin0, c_fin1])
    return logits.reshape(T, B, O), (h_n, c_n)


# speedup vs baseline: 1.1153x; 1.0173x over previous
import jax
import jax.numpy as jnp
from jax.experimental import pallas as pl
from jax.experimental.pallas import tpu as pltpu


def _sig(x):
    return 0.5 * jnp.tanh(0.5 * x) + 0.5


def _lstm_steps(wh_ref, gx_sc, hout_ref, h_sc, c_sc):
    TC = gx_sc.shape[0]
    H = h_sc.shape[1]
    h = h_sc[...]
    c = c_sc[...]
    for t in range(TC):
        g = jnp.dot(h, wh_ref[0],
                    preferred_element_type=jnp.float32) + gx_sc[t]
        sg_if = _sig(g[:, :2 * H])
        g_g = jnp.tanh(g[:, 2 * H:3 * H])
        o_g = _sig(g[:, 3 * H:])
        c = sg_if[:, H:] * c + sg_if[:, :H] * g_g
        h = o_g * jnp.tanh(c)
        hout_ref[t] = h
    h_sc[...] = h
    c_sc[...] = c


def _l0_kernel(idx_ref, emb_ref, wx_ref, wh_ref, b_ref, h0_ref, c0_ref,
               hout_ref, cfin_ref, ew_sc, gx_sc, h_sc, c_sc):
    TC, Bf, H = hout_ref.shape
    V = emb_ref.shape[0]

    @pl.when(pl.program_id(0) == 0)
    def _():
        ew_sc[...] = jnp.dot(emb_ref[...], wx_ref[0],
                             preferred_element_type=jnp.float32)
        h_sc[...] = h0_ref[0]
        c_sc[...] = c0_ref[0]

    idx = idx_ref[0]
    iota_v = jax.lax.broadcasted_iota(jnp.int32, (V, TC * Bf), 0)
    oh_t = (iota_v == idx).astype(jnp.float32)
    gx = jax.lax.dot_general(
        oh_t, ew_sc[...],
        dimension_numbers=(((0,), (0,)), ((), ())),
        preferred_element_type=jnp.float32) + b_ref[0]
    gx_sc[...] = gx.reshape(TC, Bf, 4 * H)

    _lstm_steps(wh_ref, gx_sc, hout_ref, h_sc, c_sc)
    cfin_ref[...] = c_sc[...]


def _l1_kernel(hin_ref, wx_ref, wh_ref, b_ref, h0_ref, c0_ref,
               hout_ref, cfin_ref, gx_sc, h_sc, c_sc):
    TC, Bf, H = hin_ref.shape

    @pl.when(pl.program_id(0) == 0)
    def _():
        h_sc[...] = h0_ref[0]
        c_sc[...] = c0_ref[0]

    x = hin_ref[...].reshape(TC * Bf, H)
    gx = jnp.dot(x, wx_ref[0], preferred_element_type=jnp.float32) + b_ref[0]
    gx_sc[...] = gx.reshape(TC, Bf, 4 * H)

    _lstm_steps(wh_ref, gx_sc, hout_ref, h_sc, c_sc)
    cfin_ref[...] = c_sc[...]


def _dec_kernel(x_ref, w_ref, b_ref, o_ref):
    o_ref[...] = jnp.dot(x_ref[...], w_ref[0],
                         preferred_element_type=jnp.float32) + b_ref[0]


def kernel(idx_seq, emb, w_all, b_all, h0, c0):
    T, B = idx_seq.shape
    V, H = emb.shape
    G = 4 * H
    O = 256
    TB = T * B
    TC = 16 if T % 16 == 0 else T
    NT = T // TC

    idx_r = idx_seq.astype(jnp.int32).reshape(NT, 1, TC * B)

    def layer_specs(l):
        return [
            pl.BlockSpec((1, H, G), lambda j, l=l: (l, 0, 0)),
            pl.BlockSpec((1, H, G), lambda j, l=l: (l, 1, 0)),
            pl.BlockSpec((1, 1, G), lambda j, l=l: (l, 0, 0)),
            pl.BlockSpec((1, B, H), lambda j, l=l: (l, 0, 0)),
            pl.BlockSpec((1, B, H), lambda j, l=l: (l, 0, 0)),
        ]

    out_specs = [
        pl.BlockSpec((TC, B, H), lambda j: (j, 0, 0)),
        pl.BlockSpec((B, H), lambda j: (0, 0)),
    ]
    out_shape = [jax.ShapeDtypeStruct((T, B, H), jnp.float32),
                 jax.ShapeDtypeStruct((B, H), jnp.float32)]
    state_scratch = [pltpu.VMEM((TC, B, G), jnp.float32),
                     pltpu.VMEM((B, H), jnp.float32),
                     pltpu.VMEM((B, H), jnp.float32)]
    seq_sem = pltpu.CompilerParams(dimension_semantics=("arbitrary",))

    h_all0, c_fin0 = pl.pallas_call(
        _l0_kernel,
        grid=(NT,),
        in_specs=[pl.BlockSpec((1, 1, TC * B), lambda j: (j, 0, 0)),
                  pl.BlockSpec((V, H), lambda j: (0, 0))] + layer_specs(0),
        out_specs=out_specs,
        out_shape=out_shape,
        scratch_shapes=[pltpu.VMEM((V, G), jnp.float32)] + state_scratch,
        compiler_params=seq_sem,
    )(idx_r, emb, w_all, w_all, b_all, h0, c0)

    h_all1, c_fin1 = pl.pallas_call(
        _l1_kernel,
        grid=(NT,),
        in_specs=[pl.BlockSpec((TC, B, H), lambda j: (j, 0, 0))]
                 + layer_specs(1),
        out_specs=out_specs,
        out_shape=out_shape,
        scratch_shapes=state_scratch,
        compiler_params=seq_sem,
    )(h_all0, w_all, w_all, b_all, h0, c0)

    MBd = TB // 4
    logits = pl.pallas_call(
        _dec_kernel,
        grid=(2, 2),
        in_specs=[
            pl.BlockSpec((MBd, H), lambda bi, j: (bi * 2 + j, 0)),
            pl.BlockSpec((1, H, O), lambda bi, j: (2, 0, 0)),
            pl.BlockSpec((1, 1, O), lambda bi, j: (2, 0, 0)),
        ],
        out_specs=pl.BlockSpec((MBd, O), lambda bi, j: (bi * 2 + j, 0)),
        out_shape=jax.ShapeDtypeStruct((TB, O), jnp.float32),
        compiler_params=pltpu.CompilerParams(
            dimension_semantics=("parallel", "arbitrary")),
    )(h_all1.reshape(TB, H), w_all, b_all)

    h_n = jnp.stack([h_all0[T - 1], h_all1[T - 1]])
    c_n = jnp.stack([c_fin0, c_fin1])
    return logits.reshape(T, B, O), (h_n, c_n)


# speedup vs baseline: 1.1637x; 1.0434x over previous
import jax
import jax.numpy as jnp
from jax.experimental import pallas as pl
from jax.experimental.pallas import tpu as pltpu


def _sig(x):
    return 0.5 * jnp.tanh(0.5 * x) + 0.5


def _lstm_steps(wh_ref, gx_sc, hout_ref, h_sc, c_sc):
    TC = gx_sc.shape[0]
    H = h_sc.shape[1]
    h = h_sc[...]
    c = c_sc[...]
    for t in range(TC):
        g = jnp.dot(h, wh_ref[0],
                    preferred_element_type=jnp.float32) + gx_sc[t]
        sg_if = _sig(g[:, :2 * H])
        g_g = jnp.tanh(g[:, 2 * H:3 * H])
        o_g = _sig(g[:, 3 * H:])
        c = sg_if[:, H:] * c + sg_if[:, :H] * g_g
        h = o_g * jnp.tanh(c)
        hout_ref[t] = h
    h_sc[...] = h
    c_sc[...] = c


def _l0_kernel(idx_ref, emb_ref, wx_ref, wh_ref, b_ref, h0_ref, c0_ref,
               hout_ref, cfin_ref, ew_sc, gx_sc, h_sc, c_sc):
    TC, Bf, H = hout_ref.shape
    V = emb_ref.shape[0]

    @pl.when(pl.program_id(0) == 0)
    def _():
        ew_sc[...] = jnp.dot(emb_ref[...], wx_ref[0],
                             preferred_element_type=jnp.float32)
        h_sc[...] = h0_ref[0]
        c_sc[...] = c0_ref[0]

    idx = idx_ref[0]
    iota_v = jax.lax.broadcasted_iota(jnp.int32, (V, TC * Bf), 0)
    oh_t = (iota_v == idx).astype(jnp.float32)
    gx = jax.lax.dot_general(
        oh_t, ew_sc[...],
        dimension_numbers=(((0,), (0,)), ((), ())),
        preferred_element_type=jnp.float32) + b_ref[0]
    gx_sc[...] = gx.reshape(TC, Bf, 4 * H)

    _lstm_steps(wh_ref, gx_sc, hout_ref, h_sc, c_sc)
    cfin_ref[...] = c_sc[...]


def _l1_kernel(hin_ref, wx_ref, wh_ref, b_ref, wd_ref, bd_ref,
               h0_ref, c0_ref, hout_ref, cfin_ref, logit_ref,
               gx_sc, h_sc, c_sc):
    TC, Bf, H = hin_ref.shape

    @pl.when(pl.program_id(0) == 0)
    def _():
        h_sc[...] = h0_ref[0]
        c_sc[...] = c0_ref[0]

    x = hin_ref[...].reshape(TC * Bf, H)
    gx = jnp.dot(x, wx_ref[0], preferred_element_type=jnp.float32) + b_ref[0]
    gx_sc[...] = gx.reshape(TC, Bf, 4 * H)

    _lstm_steps(wh_ref, gx_sc, hout_ref, h_sc, c_sc)
    cfin_ref[...] = c_sc[...]

    logit_ref[...] = jnp.dot(hout_ref[...].reshape(TC * Bf, H), wd_ref[0],
                             preferred_element_type=jnp.float32) + bd_ref[0]


def kernel(idx_seq, emb, w_all, b_all, h0, c0):
    T, B = idx_seq.shape
    V, H = emb.shape
    G = 4 * H
    O = 256
    TB = T * B
    TC = 16 if T % 16 == 0 else T
    NT = T // TC

    idx_r = idx_seq.astype(jnp.int32).reshape(NT, 1, TC * B)

    def layer_specs(l):
        return [
            pl.BlockSpec((1, H, G), lambda j, l=l: (l, 0, 0)),
            pl.BlockSpec((1, H, G), lambda j, l=l: (l, 1, 0)),
            pl.BlockSpec((1, 1, G), lambda j, l=l: (l, 0, 0)),
            pl.BlockSpec((1, B, H), lambda j, l=l: (l, 0, 0)),
            pl.BlockSpec((1, B, H), lambda j, l=l: (l, 0, 0)),
        ]

    out_specs = [
        pl.BlockSpec((TC, B, H), lambda j: (j, 0, 0)),
        pl.BlockSpec((B, H), lambda j: (0, 0)),
    ]
    out_shape = [jax.ShapeDtypeStruct((T, B, H), jnp.float32),
                 jax.ShapeDtypeStruct((B, H), jnp.float32)]
    state_scratch = [pltpu.VMEM((TC, B, G), jnp.float32),
                     pltpu.VMEM((B, H), jnp.float32),
                     pltpu.VMEM((B, H), jnp.float32)]
    seq_sem = pltpu.CompilerParams(dimension_semantics=("arbitrary",))

    h_all0, c_fin0 = pl.pallas_call(
        _l0_kernel,
        grid=(NT,),
        in_specs=[pl.BlockSpec((1, 1, TC * B), lambda j: (j, 0, 0)),
                  pl.BlockSpec((V, H), lambda j: (0, 0))] + layer_specs(0),
        out_specs=out_specs,
        out_shape=out_shape,
        scratch_shapes=[pltpu.VMEM((V, G), jnp.float32)] + state_scratch,
        compiler_params=seq_sem,
    )(idx_r, emb, w_all, w_all, b_all, h0, c0)

    h_all1, c_fin1, logits = pl.pallas_call(
        _l1_kernel,
        grid=(NT,),
        in_specs=[pl.BlockSpec((TC, B, H), lambda j: (j, 0, 0))]
                 + layer_specs(1)[:3]
                 + [pl.BlockSpec((1, H, O), lambda j: (2, 0, 0)),
                    pl.BlockSpec((1, 1, O), lambda j: (2, 0, 0))]
                 + layer_specs(1)[3:],
        out_specs=out_specs + [pl.BlockSpec((TC * B, O), lambda j: (j, 0))],
        out_shape=out_shape + [jax.ShapeDtypeStruct((TB, O), jnp.float32)],
        scratch_shapes=state_scratch,
        compiler_params=seq_sem,
    )(h_all0, w_all, w_all, b_all, w_all, b_all, h0, c0)

    h_n = jnp.stack([h_all0[T - 1], h_all1[T - 1]])
    c_n = jnp.stack([c_fin0, c_fin1])
    return logits.reshape(T, B, O), (h_n, c_n)


# speedup vs baseline: 1.1737x; 1.0086x over previous
import jax
import jax.numpy as jnp
from jax.experimental import pallas as pl
from jax.experimental.pallas import tpu as pltpu


def _sig(x):
    return 0.5 * jnp.tanh(0.5 * x) + 0.5


def _lstm_steps(wh_ref, gx_sc, hout_ref, h_sc, c_sc):
    TC = gx_sc.shape[0]
    H = h_sc.shape[1]
    h = h_sc[...]
    c = c_sc[...]
    for t in range(TC):
        g = jnp.dot(h, wh_ref[0],
                    preferred_element_type=jnp.float32) + gx_sc[t]
        sg_if = _sig(g[:, :2 * H])
        g_g = jnp.tanh(g[:, 2 * H:3 * H])
        o_g = _sig(g[:, 3 * H:])
        c = sg_if[:, H:] * c + sg_if[:, :H] * g_g
        h = o_g * jnp.tanh(c)
        hout_ref[t] = h
    h_sc[...] = h
    c_sc[...] = c


def _l0_kernel(idx_ref, emb_ref, wx_ref, wh_ref, b_ref, h0_ref, c0_ref,
               hout_ref, cfin_ref, ew_sc, gx_sc, h_sc, c_sc):
    TC, Bf, H = hout_ref.shape
    V = emb_ref.shape[0]

    @pl.when(pl.program_id(0) == 0)
    def _():
        ew_sc[...] = jnp.dot(emb_ref[...], wx_ref[0],
                             preferred_element_type=jnp.float32)
        h_sc[...] = h0_ref[0]
        c_sc[...] = c0_ref[0]

    idx = idx_ref[0]
    iota_v = jax.lax.broadcasted_iota(jnp.int32, (V, TC * Bf), 0)
    oh_t = (iota_v == idx).astype(jnp.float32)
    gx = jax.lax.dot_general(
        oh_t, ew_sc[...],
        dimension_numbers=(((0,), (0,)), ((), ())),
        preferred_element_type=jnp.float32) + b_ref[0]
    gx_sc[...] = gx.reshape(TC, Bf, 4 * H)

    _lstm_steps(wh_ref, gx_sc, hout_ref, h_sc, c_sc)
    cfin_ref[...] = c_sc[...]


def _l1_kernel(hin_ref, wx_ref, wh_ref, b_ref, wd_ref, bd_ref,
               h0_ref, c0_ref, hfin_ref, cfin_ref, logit_ref,
               gx_sc, hd_sc, h_sc, c_sc):
    TC, Bf, H = hin_ref.shape

    @pl.when(pl.program_id(0) == 0)
    def _():
        h_sc[...] = h0_ref[0]
        c_sc[...] = c0_ref[0]

    x = hin_ref[...].reshape(TC * Bf, H)
    gx = jnp.dot(x, wx_ref[0], preferred_element_type=jnp.float32) + b_ref[0]
    gx_sc[...] = gx.reshape(TC, Bf, 4 * H)

    _lstm_steps(wh_ref, gx_sc, hd_sc, h_sc, c_sc)
    hfin_ref[...] = h_sc[...]
    cfin_ref[...] = c_sc[...]

    logit_ref[...] = jnp.dot(hd_sc[...].reshape(TC * Bf, H), wd_ref[0],
                             preferred_element_type=jnp.float32) + bd_ref[0]


def kernel(idx_seq, emb, w_all, b_all, h0, c0):
    T, B = idx_seq.shape
    V, H = emb.shape
    G = 4 * H
    O = 256
    TB = T * B
    TC = 16 if T % 16 == 0 else T
    NT = T // TC

    idx_r = idx_seq.astype(jnp.int32).reshape(NT, 1, TC * B)

    def layer_specs(l):
        return [
            pl.BlockSpec((1, H, G), lambda j, l=l: (l, 0, 0)),
            pl.BlockSpec((1, H, G), lambda j, l=l: (l, 1, 0)),
            pl.BlockSpec((1, 1, G), lambda j, l=l: (l, 0, 0)),
            pl.BlockSpec((1, B, H), lambda j, l=l: (l, 0, 0)),
            pl.BlockSpec((1, B, H), lambda j, l=l: (l, 0, 0)),
        ]

    out_specs = [
        pl.BlockSpec((TC, B, H), lambda j: (j, 0, 0)),
        pl.BlockSpec((B, H), lambda j: (0, 0)),
    ]
    out_shape = [jax.ShapeDtypeStruct((T, B, H), jnp.float32),
                 jax.ShapeDtypeStruct((B, H), jnp.float32)]
    state_scratch = [pltpu.VMEM((TC, B, G), jnp.float32),
                     pltpu.VMEM((B, H), jnp.float32),
                     pltpu.VMEM((B, H), jnp.float32)]
    seq_sem = pltpu.CompilerParams(dimension_semantics=("arbitrary",))

    h_all0, c_fin0 = pl.pallas_call(
        _l0_kernel,
        grid=(NT,),
        in_specs=[pl.BlockSpec((1, 1, TC * B), lambda j: (j, 0, 0)),
                  pl.BlockSpec((V, H), lambda j: (0, 0))] + layer_specs(0),
        out_specs=out_specs,
        out_shape=out_shape,
        scratch_shapes=[pltpu.VMEM((V, G), jnp.float32)] + state_scratch,
        compiler_params=seq_sem,
    )(idx_r, emb, w_all, w_all, b_all, h0, c0)

    h_fin1, c_fin1, logits = pl.pallas_call(
        _l1_kernel,
        grid=(NT,),
        in_specs=[pl.BlockSpec((TC, B, H), lambda j: (j, 0, 0))]
                 + layer_specs(1)[:3]
                 + [pl.BlockSpec((1, H, O), lambda j: (2, 0, 0)),
                    pl.BlockSpec((1, 1, O), lambda j: (2, 0, 0))]
                 + layer_specs(1)[3:],
        out_specs=[pl.BlockSpec((B, H), lambda j: (0, 0)),
                   pl.BlockSpec((B, H), lambda j: (0, 0)),
                   pl.BlockSpec((TC * B, O), lambda j: (j, 0))],
        out_shape=[jax.ShapeDtypeStruct((B, H), jnp.float32),
                   jax.ShapeDtypeStruct((B, H), jnp.float32),
                   jax.ShapeDtypeStruct((TB, O), jnp.float32)],
        scratch_shapes=[pltpu.VMEM((TC, B, G), jnp.float32),
                        pltpu.VMEM((TC, B, H), jnp.float32),
                        pltpu.VMEM((B, H), jnp.float32),
                        pltpu.VMEM((B, H), jnp.float32)],
        compiler_params=seq_sem,
    )(h_all0, w_all, w_all, b_all, w_all, b_all, h0, c0)

    h_n = jnp.stack([h_all0[T - 1], h_fin1])
    c_n = jnp.stack([c_fin0, c_fin1])
    return logits.reshape(T, B, O), (h_n, c_n)


# speedup vs baseline: 1.1766x; 1.0025x over previous
import jax
import jax.numpy as jnp
from jax.experimental import pallas as pl
from jax.experimental.pallas import tpu as pltpu


def _sig(x):
    return 0.5 * jnp.tanh(0.5 * x) + 0.5


def _lstm_steps(wh_ref, gx_sc, hout_ref, h_sc, c_sc):
    TC = gx_sc.shape[0]
    H = h_sc.shape[1]
    h = h_sc[...]
    c = c_sc[...]
    for t in range(TC):
        g = jnp.dot(h, wh_ref[0],
                    preferred_element_type=jnp.float32) + gx_sc[t]
        sg_if = _sig(g[:, :2 * H])
        g_g = jnp.tanh(g[:, 2 * H:3 * H])
        o_g = _sig(g[:, 3 * H:])
        c = sg_if[:, H:] * c + sg_if[:, :H] * g_g
        h = o_g * jnp.tanh(c)
        hout_ref[t] = h.astype(hout_ref.dtype)
    h_sc[...] = h
    c_sc[...] = c


def _l0_kernel(idx_ref, emb_ref, wx_ref, wh_ref, b_ref, h0_ref, c0_ref,
               hout_ref, hfin_ref, cfin_ref, ew_sc, gx_sc, h_sc, c_sc):
    TC, Bf, H = hout_ref.shape
    V = emb_ref.shape[0]

    @pl.when(pl.program_id(0) == 0)
    def _():
        ew_sc[...] = jnp.dot(emb_ref[...], wx_ref[0],
                             preferred_element_type=jnp.float32)
        h_sc[...] = h0_ref[0]
        c_sc[...] = c0_ref[0]

    idx = idx_ref[0]
    iota_v = jax.lax.broadcasted_iota(jnp.int32, (V, TC * Bf), 0)
    oh_t = (iota_v == idx).astype(jnp.float32)
    gx = jax.lax.dot_general(
        oh_t, ew_sc[...],
        dimension_numbers=(((0,), (0,)), ((), ())),
        preferred_element_type=jnp.float32) + b_ref[0]
    gx_sc[...] = gx.reshape(TC, Bf, 4 * H)

    _lstm_steps(wh_ref, gx_sc, hout_ref, h_sc, c_sc)
    hfin_ref[...] = h_sc[...]
    cfin_ref[...] = c_sc[...]


def _l1_kernel(hin_ref, wx_ref, wh_ref, b_ref, wd_ref, bd_ref,
               h0_ref, c0_ref, hfin_ref, cfin_ref, logit_ref,
               gx_sc, hd_sc, h_sc, c_sc):
    TC, Bf, H = hin_ref.shape

    @pl.when(pl.program_id(0) == 0)
    def _():
        h_sc[...] = h0_ref[0]
        c_sc[...] = c0_ref[0]

    x = hin_ref[...].reshape(TC * Bf, H)
    gx = jax.lax.dot_general(
        x, wx_ref[0], (((1,), (0,)), ((), ())),
        preferred_element_type=jnp.float32) + b_ref[0]
    gx_sc[...] = gx.reshape(TC, Bf, 4 * H)

    _lstm_steps(wh_ref, gx_sc, hd_sc, h_sc, c_sc)
    hfin_ref[...] = h_sc[...]
    cfin_ref[...] = c_sc[...]

    logit_ref[...] = jnp.dot(hd_sc[...].reshape(TC * Bf, H), wd_ref[0],
                             preferred_element_type=jnp.float32) + bd_ref[0]


def kernel(idx_seq, emb, w_all, b_all, h0, c0):
    T, B = idx_seq.shape
    V, H = emb.shape
    G = 4 * H
    O = 256
    TB = T * B
    TC = 16 if T % 16 == 0 else T
    NT = T // TC

    idx_r = idx_seq.astype(jnp.int32).reshape(NT, 1, TC * B)

    def layer_specs(l):
        return [
            pl.BlockSpec((1, H, G), lambda j, l=l: (l, 0, 0)),
            pl.BlockSpec((1, H, G), lambda j, l=l: (l, 1, 0)),
            pl.BlockSpec((1, 1, G), lambda j, l=l: (l, 0, 0)),
            pl.BlockSpec((1, B, H), lambda j, l=l: (l, 0, 0)),
            pl.BlockSpec((1, B, H), lambda j, l=l: (l, 0, 0)),
        ]

    out_specs = [
        pl.BlockSpec((TC, B, H), lambda j: (j, 0, 0)),
        pl.BlockSpec((B, H), lambda j: (0, 0)),
        pl.BlockSpec((B, H), lambda j: (0, 0)),
    ]
    out_shape = [jax.ShapeDtypeStruct((T, B, H), jnp.bfloat16),
                 jax.ShapeDtypeStruct((B, H), jnp.float32),
                 jax.ShapeDtypeStruct((B, H), jnp.float32)]
    state_scratch = [pltpu.VMEM((TC, B, G), jnp.float32),
                     pltpu.VMEM((B, H), jnp.float32),
                     pltpu.VMEM((B, H), jnp.float32)]
    seq_sem = pltpu.CompilerParams(dimension_semantics=("arbitrary",))

    h_all0, h_fin0, c_fin0 = pl.pallas_call(
        _l0_kernel,
        grid=(NT,),
        in_specs=[pl.BlockSpec((1, 1, TC * B), lambda j: (j, 0, 0)),
                  pl.BlockSpec((V, H), lambda j: (0, 0))] + layer_specs(0),
        out_specs=out_specs,
        out_shape=out_shape,
        scratch_shapes=[pltpu.VMEM((V, G), jnp.float32)] + state_scratch,
        compiler_params=seq_sem,
    )(idx_r, emb, w_all, w_all, b_all, h0, c0)

    h_fin1, c_fin1, logits = pl.pallas_call(
        _l1_kernel,
        grid=(NT,),
        in_specs=[pl.BlockSpec((TC, B, H), lambda j: (j, 0, 0))]
                 + layer_specs(1)[:3]
                 + [pl.BlockSpec((1, H, O), lambda j: (2, 0, 0)),
                    pl.BlockSpec((1, 1, O), lambda j: (2, 0, 0))]
                 + layer_specs(1)[3:],
        out_specs=[pl.BlockSpec((B, H), lambda j: (0, 0)),
                   pl.BlockSpec((B, H), lambda j: (0, 0)),
                   pl.BlockSpec((TC * B, O), lambda j: (j, 0))],
        out_shape=[jax.ShapeDtypeStruct((B, H), jnp.float32),
                   jax.ShapeDtypeStruct((B, H), jnp.float32),
                   jax.ShapeDtypeStruct((TB, O), jnp.float32)],
        scratch_shapes=[pltpu.VMEM((TC, B, G), jnp.float32),
                        pltpu.VMEM((TC, B, H), jnp.float32),
                        pltpu.VMEM((B, H), jnp.float32),
                        pltpu.VMEM((B, H), jnp.float32)],
        compiler_params=seq_sem,
    )(h_all0, w_all, w_all, b_all, w_all, b_all, h0, c0)

    h_n = jnp.stack([h_fin0, h_fin1])
    c_n = jnp.stack([c_fin0, c_fin1])
    return logits.reshape(T, B, O), (h_n, c_n)
